# Initial kernel scaffold; baseline (speedup 1.0000x reference)
import jax
import jax.numpy as jnp
from jax.experimental import pallas as pl


def kernel(x, edge_index, edge_attr, W, b):
    raise NotImplementedError("write your pallas kernel here")


# speedup vs baseline: 4.4451x; 4.4451x over previous
import functools

import jax
import jax.numpy as jnp
from jax import lax
from jax.experimental import pallas as pl
from jax.experimental.pallas import tpu as pltpu
from jax.experimental.pallas import tpu_sc as plsc

_NC = 2
_NS = 16
_LANES = 16


def _pick_chunk(epw):
    for c in range(128, 7, -8):
        if epw % c == 0:
            return c
    raise ValueError(f"no valid chunk for {epw} edges per tile")


@functools.lru_cache(maxsize=None)
def _make_sc_scatter(n, d, e):
    n_tiles = _NC * _NS
    assert e % n_tiles == 0 and d % _LANES == 0
    epw = e // n_tiles
    chunk = _pick_chunk(epw)
    nchunk = epw // chunk
    rows_per_sub = -(-n // (8 * _NS)) * 8
    n_pad = rows_per_sub * _NS

    mesh = plsc.VectorSubcoreMesh(
        core_axis_name="c", subcore_axis_name="s",
        num_cores=_NC, num_subcores=_NS)

    @functools.partial(
        pl.kernel,
        mesh=mesh,
        out_type=jax.ShapeDtypeStruct((_NC * n_pad, d), jnp.float32),
        scratch_types=[
            pltpu.VMEM((chunk,), jnp.int32),
            pltpu.VMEM((chunk,), jnp.int32),
            pltpu.VMEM((chunk,), jnp.float32),
            pltpu.VMEM((chunk, d), jnp.float32),
            pltpu.VMEM_SHARED((n_pad, d), jnp.float32),
            pltpu.SemaphoreType.DMA,
        ],
    )
    def sc_scatter(x_hbm, src_hbm, dst_hbm, ea_hbm, zeros_hbm, out_hbm,
                   src_v, dst_v, ea_v, rows_v, acc_sh, sem):
        c = lax.axis_index("c")
        s = lax.axis_index("s")
        tid = c * _NS + s
        r0 = s * rows_per_sub

        pltpu.sync_copy(zeros_hbm, acc_sh.at[pl.ds(r0, rows_per_sub)])
        plsc.subcore_barrier()

        ebase = tid * epw

        def body(ci, carry):
            base = ebase + ci * chunk
            pltpu.sync_copy(src_hbm.at[pl.ds(base, chunk)], src_v)
            pltpu.sync_copy(dst_hbm.at[pl.ds(base, chunk)], dst_v)
            pltpu.sync_copy(ea_hbm.at[pl.ds(base, chunk)], ea_v)
            pltpu.async_copy(x_hbm.at[src_v], rows_v, sem).wait()
            for blk in range(chunk // _LANES):
                grp = ea_v[pl.ds(blk * _LANES, _LANES)]
                for j in range(_LANES):
                    i = blk * _LANES + j
                    scale = lax.gather(
                        grp, jnp.full((_LANES, 1), j, jnp.int32),
                        lax.GatherDimensionNumbers(
                            offset_dims=(), collapsed_slice_dims=(0,),
                            start_index_map=(0,)),
                        (1,), mode=lax.GatherScatterMode.PROMISE_IN_BOUNDS)
                    for g in range(d // _LANES):
                        sl = pl.ds(g * _LANES, _LANES)
                        rows_v[i, sl] = rows_v[i, sl] * scale
            pltpu.sync_copy(rows_v, acc_sh.at[dst_v], add=True)
            return carry

        lax.fori_loop(0, nchunk, body, 0)

        plsc.subcore_barrier()
        pltpu.sync_copy(acc_sh.at[pl.ds(r0, rows_per_sub)],
                        out_hbm.at[pl.ds(c * n_pad + r0, rows_per_sub)])

    return sc_scatter, n_pad, rows_per_sub


def _tc_linear(p, w, b, n):
    dout = w.shape[0]

    def mm(p_ref, w_ref, b_ref, o_ref):
        ft = p_ref[0, :n] + p_ref[1, :n]
        o_ref[...] = lax.dot_general(
            ft, w_ref[...], (((1,), (1,)), ((), ())),
            preferred_element_type=jnp.float32) + b_ref[...]

    return pl.pallas_call(
        mm,
        out_shape=jax.ShapeDtypeStruct((n, dout), jnp.float32),
    )(p, w, b.reshape(1, dout))


def kernel(x, edge_index, edge_attr, W, b):
    n, d = x.shape
    e = edge_index.shape[1]
    src = edge_index[0].astype(jnp.int32)
    dst = edge_index[1].astype(jnp.int32)
    ea = edge_attr.reshape(e).astype(jnp.float32)
    sc, n_pad, rows_per_sub = _make_sc_scatter(n, d, e)
    zeros = jnp.zeros((rows_per_sub, d), jnp.float32)
    p = sc(x, src, dst, ea, zeros)
    return _tc_linear(p.reshape(_NC, n_pad, d), W, b, n)


# speedup vs baseline: 5.7474x; 1.2930x over previous
import functools

import jax
import jax.numpy as jnp
from jax import lax
from jax.experimental import pallas as pl
from jax.experimental.pallas import tpu as pltpu
from jax.experimental.pallas import tpu_sc as plsc

_NC = 2
_NS = 16
_LANES = 16


def _pick_chunk(epw):
    for c in range(128, 7, -8):
        if epw % (2 * c) == 0:
            return c
    raise ValueError(f"no valid chunk for {epw} edges per tile")


@functools.lru_cache(maxsize=None)
def _make_sc_scatter(n, d, e):
    n_tiles = _NC * _NS
    assert e % n_tiles == 0 and d % _LANES == 0
    epw = e // n_tiles
    chunk = _pick_chunk(epw)
    nchunk = epw // chunk
    rows_per_sub = -(-n // (8 * _NS)) * 8
    n_pad = rows_per_sub * _NS

    mesh = plsc.VectorSubcoreMesh(
        core_axis_name="c", subcore_axis_name="s",
        num_cores=_NC, num_subcores=_NS)

    @functools.partial(
        pl.kernel,
        mesh=mesh,
        out_type=jax.ShapeDtypeStruct((_NC * n_pad, d), jnp.float32),
        scratch_types=[
            [pltpu.VMEM((chunk,), jnp.int32) for _ in range(2)],
            [pltpu.VMEM((chunk,), jnp.int32) for _ in range(2)],
            [pltpu.VMEM((chunk,), jnp.float32) for _ in range(2)],
            [pltpu.VMEM((chunk, d), jnp.float32) for _ in range(2)],
            pltpu.VMEM_SHARED((n_pad, d), jnp.float32),
            [pltpu.SemaphoreType.DMA for _ in range(2)],
            [pltpu.SemaphoreType.DMA for _ in range(2)],
            [pltpu.SemaphoreType.DMA for _ in range(2)],
        ],
    )
    def sc_scatter(x_hbm, src_hbm, dst_hbm, ea_hbm, zeros_hbm, out_hbm,
                   src_v, dst_v, ea_v, rows_v, acc_sh, semg, semi, sems):
        c = lax.axis_index("c")
        s = lax.axis_index("s")
        tid = c * _NS + s
        r0 = s * rows_per_sub

        pltpu.sync_copy(zeros_hbm, acc_sh.at[pl.ds(r0, rows_per_sub)])
        plsc.subcore_barrier()

        ebase = tid * epw
        e_total = e

        def load_src_ea(ci_base, b, sem):
            nb = jnp.minimum(ci_base, e_total - chunk)
            pltpu.async_copy(src_hbm.at[pl.ds(nb, chunk)], src_v[b], sem)
            pltpu.async_copy(ea_hbm.at[pl.ds(nb, chunk)], ea_v[b], sem)
            return nb

        def load_dst(nb, b, sem):
            pltpu.async_copy(dst_hbm.at[pl.ds(nb, chunk)], dst_v[b], sem)

        def drain_idx(nb, b, sem):
            pltpu.make_async_copy(
                src_hbm.at[pl.ds(nb, chunk)], src_v[b], sem).wait()
            pltpu.make_async_copy(
                ea_hbm.at[pl.ds(nb, chunk)], ea_v[b], sem).wait()
            pltpu.make_async_copy(
                dst_hbm.at[pl.ds(nb, chunk)], dst_v[b], sem).wait()

        def scale_rows(b):
            cur_g0, grp = -1, None
            for i in range(chunk):
                g0 = min((i // _LANES) * _LANES, chunk - _LANES)
                if g0 != cur_g0:
                    grp = ea_v[b][pl.ds(g0, _LANES)]
                    cur_g0 = g0
                scale = lax.gather(
                    grp, jnp.full((_LANES, 1), i - g0, jnp.int32),
                    lax.GatherDimensionNumbers(
                        offset_dims=(), collapsed_slice_dims=(0,),
                        start_index_map=(0,)),
                    (1,), mode=lax.GatherScatterMode.PROMISE_IN_BOUNDS)
                for g in range(d // _LANES):
                    sl = pl.ds(g * _LANES, _LANES)
                    rows_v[b][i, sl] = rows_v[b][i, sl] * scale

        nb0 = load_src_ea(ebase, 0, semi[0])
        load_dst(nb0, 0, semi[0])
        drain_idx(nb0, 0, semi[0])
        pltpu.async_copy(x_hbm.at[src_v[0]], rows_v[0], semg[0])
        nb1 = load_src_ea(ebase + chunk, 1, semi[1])
        load_dst(nb1, 1, semi[1])
        drain_idx(nb1, 1, semi[1])

        def body(c2, carry):
            base = ebase + 2 * c2 * chunk
            pltpu.make_async_copy(
                x_hbm.at[src_v[0]], rows_v[0], semg[0]).wait()
            pltpu.async_copy(x_hbm.at[src_v[1]], rows_v[1], semg[1])
            scale_rows(0)
            sc0 = pltpu.async_copy(
                rows_v[0], acc_sh.at[dst_v[0]], sems[0], add=True)
            nb2 = load_src_ea(base + 2 * chunk, 0, semi[0])
            pltpu.make_async_copy(
                x_hbm.at[src_v[1]], rows_v[1], semg[1]).wait()
            scale_rows(1)
            sc1 = pltpu.async_copy(
                rows_v[1], acc_sh.at[dst_v[1]], sems[1], add=True)
            sc0.wait()
            load_dst(nb2, 0, semi[0])
            drain_idx(nb2, 0, semi[0])
            pltpu.async_copy(x_hbm.at[src_v[0]], rows_v[0], semg[0])
            nb3 = load_src_ea(base + 3 * chunk, 1, semi[1])
            sc1.wait()
            load_dst(nb3, 1, semi[1])
            drain_idx(nb3, 1, semi[1])
            return carry

        lax.fori_loop(0, nchunk // 2, body, 0)
        pltpu.make_async_copy(x_hbm.at[src_v[0]], rows_v[0], semg[0]).wait()

        plsc.subcore_barrier()
        pltpu.sync_copy(acc_sh.at[pl.ds(r0, rows_per_sub)],
                        out_hbm.at[pl.ds(c * n_pad + r0, rows_per_sub)])

    return sc_scatter, n_pad, rows_per_sub


def _tc_linear(p, w, b, n):
    dout = w.shape[0]

    def mm(p_ref, w_ref, b_ref, o_ref):
        ft = p_ref[0, :n] + p_ref[1, :n]
        o_ref[...] = lax.dot_general(
            ft, w_ref[...], (((1,), (1,)), ((), ())),
            preferred_element_type=jnp.float32) + b_ref[...]

    return pl.pallas_call(
        mm,
        out_shape=jax.ShapeDtypeStruct((n, dout), jnp.float32),
    )(p, w, b.reshape(1, dout))


def kernel(x, edge_index, edge_attr, W, b):
    n, d = x.shape
    e = edge_index.shape[1]
    src = edge_index[0].astype(jnp.int32)
    dst = edge_index[1].astype(jnp.int32)
    ea = edge_attr.reshape(e).astype(jnp.float32)
    sc, n_pad, rows_per_sub = _make_sc_scatter(n, d, e)
    zeros = jnp.zeros((rows_per_sub, d), jnp.float32)
    p = sc(x, src, dst, ea, zeros)
    return _tc_linear(p.reshape(_NC, n_pad, d), W, b, n)


# speedup vs baseline: 8.5322x; 1.4845x over previous
import functools

import jax
import jax.numpy as jnp
from jax import lax
from jax.experimental import pallas as pl
from jax.experimental.pallas import tpu as pltpu
from jax.experimental.pallas import tpu_sc as plsc

_NC = 2
_NS = 16
_LANES = 16


_RING = 4


def _pick_chunk(epw):
    for c in range(128, 7, -8):
        if epw % c == 0 and epw // c >= 2 * _RING:
            return c
    raise ValueError(f"no valid chunk for {epw} edges per tile")


@functools.lru_cache(maxsize=None)
def _make_sc_scatter(n, d, e):
    n_tiles = _NC * _NS
    assert e % n_tiles == 0 and d % _LANES == 0
    epw = e // n_tiles
    chunk = _pick_chunk(epw)
    nchunk = epw // chunk
    rows_per_sub = -(-n // (8 * _NS)) * 8
    n_pad = rows_per_sub * _NS

    mesh = plsc.VectorSubcoreMesh(
        core_axis_name="c", subcore_axis_name="s",
        num_cores=_NC, num_subcores=_NS)

    @functools.partial(
        pl.kernel,
        mesh=mesh,
        out_type=jax.ShapeDtypeStruct((_NC * n_pad, d), jnp.float32),
        scratch_types=[
            [pltpu.VMEM((chunk,), jnp.int32) for _ in range(_RING)],
            [pltpu.VMEM((chunk,), jnp.int32) for _ in range(_RING)],
            [pltpu.VMEM((chunk,), jnp.float32) for _ in range(_RING)],
            [pltpu.VMEM((chunk, d), jnp.float32) for _ in range(_RING)],
            pltpu.VMEM_SHARED((n_pad, d), jnp.float32),
            [pltpu.SemaphoreType.DMA for _ in range(_RING)],
            [pltpu.SemaphoreType.DMA for _ in range(_RING)],
            [pltpu.SemaphoreType.DMA for _ in range(_RING)],
        ],
    )
    def sc_scatter(x_hbm, src_hbm, dst_hbm, ea_hbm, zeros_hbm, out_hbm,
                   src_v, dst_v, ea_v, rows_v, acc_sh, semg, semi, sems):
        c = lax.axis_index("c")
        s = lax.axis_index("s")
        tid = c * _NS + s
        r0 = s * rows_per_sub

        pltpu.sync_copy(zeros_hbm, acc_sh.at[pl.ds(r0, rows_per_sub)])
        plsc.subcore_barrier()

        ebase = tid * epw
        e_total = e

        def load_idx(ci_base, b):
            nb = jnp.minimum(ci_base, e_total - chunk)
            pltpu.async_copy(src_hbm.at[pl.ds(nb, chunk)], src_v[b], semi[b])
            pltpu.async_copy(dst_hbm.at[pl.ds(nb, chunk)], dst_v[b], semi[b])
            pltpu.async_copy(ea_hbm.at[pl.ds(nb, chunk)], ea_v[b], semi[b])
            return nb

        def drain_idx(nb, b):
            pltpu.make_async_copy(
                src_hbm.at[pl.ds(nb, chunk)], src_v[b], semi[b]).wait()
            pltpu.make_async_copy(
                dst_hbm.at[pl.ds(nb, chunk)], dst_v[b], semi[b]).wait()
            pltpu.make_async_copy(
                ea_hbm.at[pl.ds(nb, chunk)], ea_v[b], semi[b]).wait()

        def gather(b):
            pltpu.async_copy(x_hbm.at[src_v[b]], rows_v[b], semg[b])

        def wait_gather(b):
            pltpu.make_async_copy(
                x_hbm.at[src_v[b]], rows_v[b], semg[b]).wait()

        def scatter(b):
            pltpu.async_copy(
                rows_v[b], acc_sh.at[dst_v[b]], sems[b], add=True)

        def wait_scatter(b):
            pltpu.make_async_copy(
                rows_v[b], acc_sh.at[dst_v[b]], sems[b]).wait()

        def scale_rows(b):
            cur_g0, grp = -1, None
            for i in range(chunk):
                g0 = min((i // _LANES) * _LANES, chunk - _LANES)
                if g0 != cur_g0:
                    grp = ea_v[b][pl.ds(g0, _LANES)]
                    cur_g0 = g0
                scale = lax.gather(
                    grp, jnp.full((_LANES, 1), i - g0, jnp.int32),
                    lax.GatherDimensionNumbers(
                        offset_dims=(), collapsed_slice_dims=(0,),
                        start_index_map=(0,)),
                    (1,), mode=lax.GatherScatterMode.PROMISE_IN_BOUNDS)
                for g in range(d // _LANES):
                    sl = pl.ds(g * _LANES, _LANES)
                    rows_v[b][i, sl] = rows_v[b][i, sl] * scale

        p_len = _RING + (nchunk % _RING)
        nbs = [load_idx(ebase + q * chunk, q % _RING)
               for q in range(min(_RING, p_len))]
        for q in range(min(_RING, p_len)):
            drain_idx(nbs[q], q % _RING)
        for q in range(p_len):
            b = q % _RING
            if q >= _RING:
                wait_scatter(b)
                nb = load_idx(ebase + q * chunk, b)
                drain_idx(nb, b)
            gather(b)
            wait_gather(b)
            scale_rows(b)
            scatter(b)
        for q in (p_len - _RING, p_len - _RING + 1):
            wait_scatter(q % _RING)
        nb = load_idx(ebase + p_len * chunk, p_len % _RING)
        drain_idx(nb, p_len % _RING)
        gather(p_len % _RING)
        nb = load_idx(ebase + (p_len + 1) * chunk, (p_len + 1) % _RING)
        drain_idx(nb, (p_len + 1) % _RING)

        def body(ci, carry):
            qbase = ebase + (p_len + ci * _RING) * chunk
            for j in range(_RING):
                b = (p_len + j) % _RING
                b1, b2 = (b + 1) % _RING, (b + 2) % _RING
                wait_scatter(b2)
                nb2 = load_idx(qbase + (j + 2) * chunk, b2)
                wait_gather(b)
                gather(b1)
                scale_rows(b)
                scatter(b)
                drain_idx(nb2, b2)
            return carry

        lax.fori_loop(0, (nchunk - p_len) // _RING, body, 0)
        wait_gather(nchunk % _RING)
        for q in range(nchunk - 2, nchunk):
            wait_scatter(q % _RING)

        plsc.subcore_barrier()
        pltpu.sync_copy(acc_sh.at[pl.ds(r0, rows_per_sub)],
                        out_hbm.at[pl.ds(c * n_pad + r0, rows_per_sub)])

    return sc_scatter, n_pad, rows_per_sub


def _tc_linear(p, w, b, n):
    dout = w.shape[0]

    def mm(p_ref, w_ref, b_ref, o_ref):
        ft = p_ref[0, :n] + p_ref[1, :n]
        o_ref[...] = lax.dot_general(
            ft, w_ref[...], (((1,), (1,)), ((), ())),
            preferred_element_type=jnp.float32) + b_ref[...]

    return pl.pallas_call(
        mm,
        out_shape=jax.ShapeDtypeStruct((n, dout), jnp.float32),
    )(p, w, b.reshape(1, dout))


def kernel(x, edge_index, edge_attr, W, b):
    n, d = x.shape
    e = edge_index.shape[1]
    src = edge_index[0].astype(jnp.int32)
    dst = edge_index[1].astype(jnp.int32)
    ea = edge_attr.reshape(e).astype(jnp.float32)
    sc, n_pad, rows_per_sub = _make_sc_scatter(n, d, e)
    zeros = jnp.zeros((rows_per_sub, d), jnp.float32)
    p = sc(x, src, dst, ea, zeros)
    return _tc_linear(p.reshape(_NC, n_pad, d), W, b, n)
